# baseline (speedup 1.0000x reference)
import jax
import jax.numpy as jnp
from jax import lax
from jax.experimental import pallas as pl
from jax.experimental.pallas import tpu as pltpu
from jax.experimental.pallas import tpu_sc as plsc

_NC, _NS, _L = 2, 16, 16
_NW = _NC * _NS

_B, _SEQ, _D = 4, 4096, 1024
_ROWS_PER_W = _SEQ // _NW
_CH = 16
_NCH = _ROWS_PER_W // _CH
_NXB = 4


def _sc_body(x_hbm, pos_hbm, out_hbm,
             xv0, xv1, xv2, xv3, pv0, pv1, idxv,
             xs0, xs1, xs2, xs3, os0, os1, os2, os3, ps0, ps1):
    wid = lax.axis_index("s") * _NC + lax.axis_index("c")
    base_s = wid * _ROWS_PER_W
    del idxv
    xbufs = [xv0, xv1, xv2, xv3]
    pbufs = [pv0, pv1]
    xsems = [xs0, xs1, xs2, xs3]
    osems = [os0, os1, os2, os3]
    psems = [ps0, ps1]

    units = [(c, b) for c in range(_NCH) for b in range(_B)]

    def x_slice(u):
        c, b = units[u]
        return x_hbm.at[b, pl.ds(base_s + c * _CH, _CH)]

    pos_cp = [None, None]
    x_cp = [None] * _NXB
    st_cp = [None] * _NXB

    pos_cp[0] = pltpu.async_copy(pos_hbm.at[pl.ds(base_s, _CH)], pv0, ps0)
    for u in range(_NXB - 1):
        x_cp[u] = pltpu.async_copy(x_slice(u), xbufs[u], xsems[u])

    for i, (c, b) in enumerate(units):
        nb = i % _NXB
        nxt = i + _NXB - 1
        if nxt < len(units):
            onb = nxt % _NXB
            if st_cp[onb] is not None:
                st_cp[onb].wait()
                st_cp[onb] = None
            x_cp[onb] = pltpu.async_copy(x_slice(nxt), xbufs[onb], xsems[onb])
        if b == 0 and c + 1 < _NCH:
            pos_cp[(c + 1) % 2] = pltpu.async_copy(
                pos_hbm.at[pl.ds(base_s + (c + 1) * _CH, _CH)],
                pbufs[(c + 1) % 2], psems[(c + 1) % 2])
        if b == 0:
            pos_cp[c % 2].wait()
        x_cp[nb].wait()

        xv, pv = xbufs[nb], pbufs[c % 2]

        @plsc.parallel_loop(0, _D // _L, unroll=1)
        def col_body(j):
            sl = pl.ds(j * _L, _L)
            for r in range(_CH):
                plsc.addupdate(xv.at[r, sl], pv[r, sl])
        st_cp[nb] = pltpu.async_copy(
            xv, out_hbm.at[b, pl.ds(base_s + c * _CH, _CH)], osems[nb])

    for cp in st_cp:
        if cp is not None:
            cp.wait()


def kernel(x, pos_table):
    batch, seq_len, d_model = x.shape
    k = pl.kernel(
        _sc_body,
        out_type=jax.ShapeDtypeStruct((batch, seq_len, d_model), x.dtype),
        mesh=plsc.VectorSubcoreMesh(core_axis_name="c", subcore_axis_name="s"),
        scratch_types=(
            [pltpu.VMEM((_CH, _D), jnp.float32)] * (_NXB + 2)
            + [pltpu.VMEM((_CH,), jnp.int32)]
            + [pltpu.SemaphoreType.DMA] * (2 * _NXB + 2)
        ),
    )
    return k(x, pos_table)


# speedup vs baseline: 1.0971x; 1.0971x over previous
import jax
import jax.numpy as jnp
from jax import lax
from jax.experimental import pallas as pl
from jax.experimental.pallas import tpu as pltpu
from jax.experimental.pallas import tpu_sc as plsc

_NC, _NS, _L = 2, 16, 16
_NW = _NC * _NS

_B, _SEQ, _D = 4, 4096, 1024
_ROWS_PER_W = _SEQ // _NW
_CH = 8
_NCH = _ROWS_PER_W // _CH
_NSET = 3


def _sc_body(x_hbm, pos_hbm, out_hbm, *refs):
    xv = refs[0:_NSET * _B]
    pv = refs[_NSET * _B:_NSET * _B + 2]
    xls = refs[_NSET * _B + 2:2 * _NSET * _B + 2]
    xss = refs[2 * _NSET * _B + 2:3 * _NSET * _B + 2]
    pss = refs[3 * _NSET * _B + 2:]

    wid = lax.axis_index("s") * _NC + lax.axis_index("c")
    base_s = wid * _ROWS_PER_W

    pos_cp = [None, None]
    x_cp = [None] * (_NSET * _B)
    st_cp = [None] * (_NSET * _B)

    pos_cp[0] = pltpu.async_copy(pos_hbm.at[pl.ds(base_s, _CH)], pv[0], pss[0])
    for b in range(_B):
        x_cp[b] = pltpu.async_copy(
            x_hbm.at[b, pl.ds(base_s, _CH)], xv[b], xls[b])

    for c in range(_NCH):
        s = c % _NSET
        off = base_s + c * _CH
        if c + 1 < _NCH:
            o = (c + 1) % _NSET
            for b in range(_B):
                k = o * _B + b
                if st_cp[k] is not None:
                    st_cp[k].wait()
                    st_cp[k] = None
                x_cp[k] = pltpu.async_copy(
                    x_hbm.at[b, pl.ds(off + _CH, _CH)], xv[k], xls[k])
            pos_cp[(c + 1) % 2] = pltpu.async_copy(
                pos_hbm.at[pl.ds(off + _CH, _CH)],
                pv[(c + 1) % 2], pss[(c + 1) % 2])
        pos_cp[c % 2].wait()
        for b in range(_B):
            x_cp[s * _B + b].wait()

        bufs = [xv[s * _B + b] for b in range(_B)]
        pvv = pv[c % 2]

        @plsc.parallel_loop(0, _D // _L, unroll=1)
        def col_body(j):
            sl = pl.ds(j * _L, _L)
            for r in range(_CH):
                p = pvv[r, sl]
                for xb in bufs:
                    xb[r, sl] = xb[r, sl] + p

        for b in range(_B):
            k = s * _B + b
            st_cp[k] = pltpu.async_copy(
                xv[k], out_hbm.at[b, pl.ds(off, _CH)], xss[k])

    for cp in st_cp:
        if cp is not None:
            cp.wait()


def kernel(x, pos_table):
    batch, seq_len, d_model = x.shape
    k = pl.kernel(
        _sc_body,
        out_type=jax.ShapeDtypeStruct((batch, seq_len, d_model), x.dtype),
        mesh=plsc.VectorSubcoreMesh(core_axis_name="c", subcore_axis_name="s"),
        scratch_types=(
            [pltpu.VMEM((_CH, _D), jnp.float32)] * (_NSET * _B)
            + [pltpu.VMEM((_CH, _D), jnp.float32)] * 2
            + [pltpu.SemaphoreType.DMA] * (2 * _NSET * _B)
            + [pltpu.SemaphoreType.DMA] * 2
        ),
    )
    return k(x, pos_table)
